# Initial kernel scaffold; baseline (speedup 1.0000x reference)
import functools

import jax
import jax.numpy as jnp
from jax import lax
from jax.experimental import pallas as pl
from jax.experimental.pallas import tpu as pltpu
from jax.experimental.pallas import tpu_sc as plsc

VOCAB = 1000000
EMBED = 16
SEQLEN = 200
BATCH = 16384
HIDDEN = 24

NC = 2
NS = 16
NW = NC * NS
BPW = BATCH // NW
CHUNK = 100
CPR = SEQLEN // CHUNK
NCH = BPW * CPR
NBUF = 4
NACC = 8


def _pool_body(idx_hbm, table_hbm, out_hbm,
               idx_v, b0, b1, b2, b3, out_v, s0, s1, s2, s3):
    bufs = (b0, b1, b2, b3)
    sems = (s0, s1, s2, s3)
    c = lax.axis_index("c")
    s = lax.axis_index("s")
    wid = s * NC + c

    pltpu.sync_copy(idx_hbm.at[wid], idx_v)

    def _gather(g, b):
        return pltpu.async_copy(table_hbm.at[idx_v.at[g]], bufs[b], sems[b])

    for b in range(NBUF):
        _gather(b, b)

    def _sum_buf(b):
        parts = [bufs[b][j, :] for j in range(NACC)]
        for j in range(NACC, CHUNK):
            parts[j % NACC] = parts[j % NACC] + bufs[b][j, :]
        while len(parts) > 1:
            parts = [parts[k] + parts[k + 1] for k in range(0, len(parts), 2)]
        return parts[0]

    def _step(i, carry):
        g0 = i * NBUF
        accs = []
        for b in range(NBUF):
            g = g0 + b
            pltpu.make_async_copy(table_hbm.at[idx_v.at[g]], bufs[b], sems[b]).wait()
            accs.append(_sum_buf(b))
            nxt = g + NBUF

            @pl.when(nxt < NCH)
            def _():
                _gather(nxt, b)

        row = 2 * i
        out_v[row, :] = accs[0] + accs[1]
        out_v[row + 1, :] = accs[2] + accs[3]
        return carry

    lax.fori_loop(0, NCH // NBUF, _step, 0, unroll=False)

    pltpu.sync_copy(out_v, out_hbm.at[pl.ds(wid * BPW, BPW)])


@jax.jit
def _pool(idx, table):
    mesh = plsc.VectorSubcoreMesh(
        core_axis_name="c", subcore_axis_name="s",
        num_cores=NC, num_subcores=NS)
    f = pl.kernel(
        _pool_body,
        out_type=jax.ShapeDtypeStruct((BATCH, EMBED), jnp.float32),
        mesh=mesh,
        scratch_types=[
            pltpu.VMEM((NCH, CHUNK), jnp.int32),
            pltpu.VMEM((CHUNK, EMBED), jnp.float32),
            pltpu.VMEM((CHUNK, EMBED), jnp.float32),
            pltpu.VMEM((CHUNK, EMBED), jnp.float32),
            pltpu.VMEM((CHUNK, EMBED), jnp.float32),
            pltpu.VMEM((BPW, EMBED), jnp.float32),
            pltpu.SemaphoreType.DMA,
            pltpu.SemaphoreType.DMA,
            pltpu.SemaphoreType.DMA,
            pltpu.SemaphoreType.DMA,
        ],
    )
    return f(idx, table)


def _mlp_body(x_ref, w1_ref, b1_ref, w2_ref, b2_ref, o_ref):
    x = x_ref[...]
    h = jnp.dot(x, w1_ref[...], preferred_element_type=jnp.float32) + b1_ref[...]
    h = jnp.maximum(h, 0.0)
    z = jnp.dot(h, w2_ref[...], preferred_element_type=jnp.float32) + b2_ref[...]
    o_ref[...] = 1.0 / (1.0 + jnp.exp(-z))


@jax.jit
def _mlp(pooled, w1s, b1, w2, b2):
    return pl.pallas_call(
        _mlp_body,
        out_shape=jax.ShapeDtypeStruct((BATCH, 1), jnp.float32),
    )(pooled, w1s, b1, w2, b2)


def kernel(indices, table, W1, b1, W2, b2):
    idx = indices.astype(jnp.int32).reshape(NW, NCH, CHUNK)
    pooled = _pool(idx, table)
    return _mlp(pooled, W1 * (1.0 / SEQLEN), b1.reshape(1, HIDDEN),
                W2, b2.reshape(1, 1))


# speedup vs baseline: 7.6700x; 7.6700x over previous
import functools

import jax
import jax.numpy as jnp
from jax import lax
from jax.experimental import pallas as pl
from jax.experimental.pallas import tpu as pltpu
from jax.experimental.pallas import tpu_sc as plsc

VOCAB = 1000000
EMBED = 16
SEQLEN = 200
BATCH = 16384
HIDDEN = 24

NC = 2
NS = 16
NW = NC * NS
BPW = BATCH // NW
CHUNK = 100
CPR = SEQLEN // CHUNK
NCH = BPW * CPR
NBUF = 4
NACC = 8


def _pool_body(idx_hbm, table_hbm, out_hbm,
               idx_v, b0, b1, b2, b3, out_v, s0, s1, s2, s3):
    bufs = (b0, b1, b2, b3)
    sems = (s0, s1, s2, s3)
    c = lax.axis_index("c")
    s = lax.axis_index("s")
    wid = s * NC + c

    pltpu.sync_copy(idx_hbm.at[wid], idx_v)

    def _gather(g, b):
        return pltpu.async_copy(table_hbm.at[idx_v.at[g]], bufs[b], sems[b])

    for b in range(NBUF):
        _gather(b, b)

    def _sum_buf(b):
        parts = [bufs[b][j, :] for j in range(NACC)]
        for j in range(NACC, CHUNK):
            parts[j % NACC] = parts[j % NACC] + bufs[b][j, :]
        while len(parts) > 1:
            parts = [parts[k] + parts[k + 1] for k in range(0, len(parts), 2)]
        return parts[0]

    def _step(i, carry):
        g0 = i * NBUF
        accs = []
        for b in range(NBUF):
            g = g0 + b
            pltpu.make_async_copy(table_hbm.at[idx_v.at[g]], bufs[b], sems[b]).wait()
            accs.append(_sum_buf(b))
            nxt = g + NBUF

            @pl.when(nxt < NCH)
            def _():
                _gather(nxt, b)

        row = 2 * i
        out_v[row, :] = accs[0] + accs[1]
        out_v[row + 1, :] = accs[2] + accs[3]
        return carry

    lax.fori_loop(0, NCH // NBUF, _step, 0, unroll=False)

    pltpu.sync_copy(out_v, out_hbm.at[pl.ds(wid * BPW, BPW)])


@jax.jit
def _pool(idx, table):
    mesh = plsc.VectorSubcoreMesh(
        core_axis_name="c", subcore_axis_name="s",
        num_cores=NC, num_subcores=NS)
    f = pl.kernel(
        _pool_body,
        out_type=jax.ShapeDtypeStruct((BATCH, EMBED), jnp.float32),
        mesh=mesh,
        compiler_params=pltpu.CompilerParams(use_tc_tiling_on_sc=False),
        scratch_types=[
            pltpu.VMEM((NCH, CHUNK), jnp.int32),
            pltpu.VMEM((CHUNK, EMBED), jnp.float32),
            pltpu.VMEM((CHUNK, EMBED), jnp.float32),
            pltpu.VMEM((CHUNK, EMBED), jnp.float32),
            pltpu.VMEM((CHUNK, EMBED), jnp.float32),
            pltpu.VMEM((BPW, EMBED), jnp.float32),
            pltpu.SemaphoreType.DMA,
            pltpu.SemaphoreType.DMA,
            pltpu.SemaphoreType.DMA,
            pltpu.SemaphoreType.DMA,
        ],
    )
    return f(idx, table)


def _mlp_body(x_ref, w1_ref, b1_ref, w2_ref, b2_ref, o_ref):
    x = x_ref[...]
    h = jnp.dot(x, w1_ref[...], preferred_element_type=jnp.float32) + b1_ref[...]
    h = jnp.maximum(h, 0.0)
    z = jnp.dot(h, w2_ref[...], preferred_element_type=jnp.float32) + b2_ref[...]
    o_ref[...] = 1.0 / (1.0 + jnp.exp(-z))


@jax.jit
def _mlp(pooled, w1s, b1, w2, b2):
    return pl.pallas_call(
        _mlp_body,
        out_shape=jax.ShapeDtypeStruct((BATCH, 1), jnp.float32),
    )(pooled, w1s, b1, w2, b2)


def kernel(indices, table, W1, b1, W2, b2):
    idx = indices.astype(jnp.int32).reshape(NW, NCH, CHUNK)
    pooled = _pool(idx, table)
    return _mlp(pooled, W1 * (1.0 / SEQLEN), b1.reshape(1, HIDDEN),
                W2, b2.reshape(1, 1))


# speedup vs baseline: 9.2380x; 1.2044x over previous
import jax
import jax.numpy as jnp
from jax import lax
from jax.experimental import pallas as pl
from jax.experimental.pallas import tpu as pltpu
from jax.experimental.pallas import tpu_sc as plsc

VOCAB = 1000000
EMBED = 16
SEQLEN = 200
BATCH = 16384
HIDDEN = 24

NC = 2
NS = 16
NW = NC * NS
BPW = BATCH // NW
RPS = 16
NSTEP = BPW // RPS
IPS = RPS * SEQLEN
GCH = 128
NCHS = IPS // GCH
NACC = 8


def _pool_body(idx_hbm, table_hbm, out_hbm,
               i0, i1, g0, g1, out_v, si0, si1, sg0, sg1):
    idxs = (i0, i1)
    gbufs = (g0, g1)
    isems = (si0, si1)
    gsems = (sg0, sg1)
    c = lax.axis_index("c")
    s = lax.axis_index("s")
    wid = s * NC + c

    def idx_copy(st, p):
        pltpu.async_copy(idx_hbm.at[wid, st], idxs[p], isems[p])

    def idx_wait(p):
        pltpu.make_async_copy(idx_hbm.at[wid, 0], idxs[p], isems[p]).wait()

    def gathers(p):
        for ch in range(NCHS):
            sl = pl.ds(ch * GCH, GCH)
            pltpu.async_copy(table_hbm.at[idxs[p].at[sl]],
                             gbufs[p].at[sl], gsems[p])

    def gathers_wait(p):
        pltpu.make_async_copy(table_hbm.at[idxs[p]], gbufs[p], gsems[p]).wait()

    def accum(p, st):
        buf = gbufs[p]

        def row_body(r, carry):
            base = r * SEQLEN
            parts = [buf[base + j, :] for j in range(NACC)]
            for j in range(NACC, SEQLEN):
                parts[j % NACC] = parts[j % NACC] + buf[base + j, :]
            while len(parts) > 1:
                parts = [parts[k] + parts[k + 1]
                         for k in range(0, len(parts), 2)]
            out_v[st * RPS + r, :] = parts[0]
            return carry

        lax.fori_loop(0, RPS, row_body, 0, unroll=False)

    idx_copy(0, 0)
    idx_wait(0)
    gathers(0)
    idx_copy(1, 1)

    def outer(s2, carry):
        for p in range(2):
            st = 2 * s2 + p
            gathers_wait(p)

            @pl.when(st + 2 < NSTEP)
            def _():
                idx_copy(st + 2, p)

            @pl.when(st + 1 < NSTEP)
            def _():
                idx_wait(1 - p)
                gathers(1 - p)

            accum(p, st)
        return carry

    lax.fori_loop(0, NSTEP // 2, outer, 0, unroll=False)

    pltpu.sync_copy(out_v, out_hbm.at[pl.ds(wid * BPW, BPW)])


@jax.jit
def _pool(idx, table):
    mesh = plsc.VectorSubcoreMesh(
        core_axis_name="c", subcore_axis_name="s",
        num_cores=NC, num_subcores=NS)
    f = pl.kernel(
        _pool_body,
        out_type=jax.ShapeDtypeStruct((BATCH, EMBED), jnp.float32),
        mesh=mesh,
        compiler_params=pltpu.CompilerParams(use_tc_tiling_on_sc=False),
        scratch_types=[
            pltpu.VMEM((IPS,), jnp.int32),
            pltpu.VMEM((IPS,), jnp.int32),
            pltpu.VMEM((IPS, EMBED), jnp.float32),
            pltpu.VMEM((IPS, EMBED), jnp.float32),
            pltpu.VMEM((BPW, EMBED), jnp.float32),
            pltpu.SemaphoreType.DMA,
            pltpu.SemaphoreType.DMA,
            pltpu.SemaphoreType.DMA,
            pltpu.SemaphoreType.DMA,
        ],
    )
    return f(idx, table)


def _mlp_body(x_ref, w1_ref, b1_ref, w2_ref, b2_ref, o_ref):
    x = x_ref[...]
    h = jnp.dot(x, w1_ref[...], preferred_element_type=jnp.float32) + b1_ref[...]
    h = jnp.maximum(h, 0.0)
    z = jnp.dot(h, w2_ref[...], preferred_element_type=jnp.float32) + b2_ref[...]
    o_ref[...] = 1.0 / (1.0 + jnp.exp(-z))


@jax.jit
def _mlp(pooled, w1s, b1, w2, b2):
    return pl.pallas_call(
        _mlp_body,
        out_shape=jax.ShapeDtypeStruct((BATCH, 1), jnp.float32),
    )(pooled, w1s, b1, w2, b2)


def kernel(indices, table, W1, b1, W2, b2):
    idx = indices.astype(jnp.int32).reshape(NW, NSTEP, IPS)
    pooled = _pool(idx, table)
    return _mlp(pooled, W1 * (1.0 / SEQLEN), b1.reshape(1, HIDDEN),
                W2, b2.reshape(1, 1))


# speedup vs baseline: 9.2381x; 1.0000x over previous
import jax
import jax.numpy as jnp
from jax import lax
from jax.experimental import pallas as pl
from jax.experimental.pallas import tpu as pltpu
from jax.experimental.pallas import tpu_sc as plsc

VOCAB = 1000000
EMBED = 16
SEQLEN = 200
BATCH = 16384
HIDDEN = 24

NC = 2
NS = 16
NW = NC * NS
BPW = BATCH // NW
RPS = 16
NSTEP = BPW // RPS
IPS = RPS * SEQLEN
SPLITS = (0, 104)
NACC = 8


def _pool_body(idx_hbm, table_hbm, out_hbm,
               i0, i1, g0, g1, out_v, si0, si1, sg0, sg1):
    idxs = (i0, i1)
    gbufs = (g0, g1)
    isems = (si0, si1)
    gsems = (sg0, sg1)
    c = lax.axis_index("c")
    s = lax.axis_index("s")
    wid = s * NC + c
    row0 = wid * BPW

    def idx_copy(st, p):
        pltpu.async_copy(idx_hbm.at[pl.ds(row0 + st * RPS, RPS), :],
                         idxs[p], isems[p])

    def idx_wait(p):
        pltpu.make_async_copy(idx_hbm.at[pl.ds(0, RPS), :], idxs[p],
                              isems[p]).wait()

    def gathers(p):
        def gather_row(r, carry):
            for k, off in enumerate(SPLITS):
                n = (SPLITS[k + 1] if k + 1 < len(SPLITS) else SEQLEN) - off
                pltpu.async_copy(
                    table_hbm.at[idxs[p].at[r, pl.ds(off, n)]],
                    gbufs[p].at[pl.ds(r * SEQLEN + off, n)], gsems[p])
            return carry

        lax.fori_loop(0, RPS, gather_row, 0, unroll=False)

    def gathers_wait(p):
        pltpu.make_async_copy(out_hbm.at[pl.ds(0, IPS)], gbufs[p],
                              gsems[p]).wait()

    def accum(p, st):
        buf = gbufs[p]

        def row_body(r, carry):
            base = r * SEQLEN
            parts = [buf[base + j, :] for j in range(NACC)]
            for j in range(NACC, SEQLEN):
                parts[j % NACC] = parts[j % NACC] + buf[base + j, :]
            while len(parts) > 1:
                parts = [parts[k] + parts[k + 1]
                         for k in range(0, len(parts), 2)]
            out_v[st * RPS + r, :] = parts[0]
            return carry

        lax.fori_loop(0, RPS, row_body, 0, unroll=False)

    idx_copy(0, 0)
    idx_wait(0)
    gathers(0)
    idx_copy(1, 1)

    def outer(s2, carry):
        for p in range(2):
            st = 2 * s2 + p
            gathers_wait(p)

            @pl.when(st + 2 < NSTEP)
            def _():
                idx_copy(st + 2, p)

            @pl.when(st + 1 < NSTEP)
            def _():
                idx_wait(1 - p)
                gathers(1 - p)

            accum(p, st)
        return carry

    lax.fori_loop(0, NSTEP // 2, outer, 0, unroll=False)

    pltpu.sync_copy(out_v, out_hbm.at[pl.ds(wid * BPW, BPW)])


@jax.jit
def _pool(idx, table):
    mesh = plsc.VectorSubcoreMesh(
        core_axis_name="c", subcore_axis_name="s",
        num_cores=NC, num_subcores=NS)
    f = pl.kernel(
        _pool_body,
        out_type=jax.ShapeDtypeStruct((BATCH, EMBED), jnp.float32),
        mesh=mesh,
        compiler_params=pltpu.CompilerParams(use_tc_tiling_on_sc=False),
        scratch_types=[
            pltpu.VMEM((RPS, SEQLEN), jnp.int32),
            pltpu.VMEM((RPS, SEQLEN), jnp.int32),
            pltpu.VMEM((IPS, EMBED), jnp.float32),
            pltpu.VMEM((IPS, EMBED), jnp.float32),
            pltpu.VMEM((BPW, EMBED), jnp.float32),
            pltpu.SemaphoreType.DMA,
            pltpu.SemaphoreType.DMA,
            pltpu.SemaphoreType.DMA,
            pltpu.SemaphoreType.DMA,
        ],
    )
    return f(idx, table)


def _mlp_body(x_ref, w1_ref, b1_ref, w2_ref, b2_ref, o_ref):
    x = x_ref[...]
    h = jnp.dot(x, w1_ref[...], preferred_element_type=jnp.float32) + b1_ref[...]
    h = jnp.maximum(h, 0.0)
    z = jnp.dot(h, w2_ref[...], preferred_element_type=jnp.float32) + b2_ref[...]
    o_ref[...] = 1.0 / (1.0 + jnp.exp(-z))


@jax.jit
def _mlp(pooled, w1s, b1, w2, b2):
    return pl.pallas_call(
        _mlp_body,
        out_shape=jax.ShapeDtypeStruct((BATCH, 1), jnp.float32),
    )(pooled, w1s, b1, w2, b2)


def kernel(indices, table, W1, b1, W2, b2):
    idx = indices.astype(jnp.int32)
    pooled = _pool(idx, table)
    return _mlp(pooled, W1 * (1.0 / SEQLEN), b1.reshape(1, HIDDEN),
                W2, b2.reshape(1, 1))


# speedup vs baseline: 9.2428x; 1.0005x over previous
import jax
import jax.numpy as jnp
from jax import lax
from jax.experimental import pallas as pl
from jax.experimental.pallas import tpu as pltpu
from jax.experimental.pallas import tpu_sc as plsc

VOCAB = 1000000
EMBED = 16
SEQLEN = 200
BATCH = 16384
HIDDEN = 24

NC = 2
NS = 16
NW = NC * NS
BPW = BATCH // NW
RPS = 16
NSTEP = BPW // RPS
IPS = RPS * SEQLEN
GCH = 128
NCHS = IPS // GCH
CPW = (BPW * SEQLEN) // GCH
NACC = 8


def _pool_body(idx_hbm, table_hbm, out_hbm,
               i0, i1, g0, g1, out_v, si0, si1, sg0, sg1):
    idxs = (i0, i1)
    gbufs = (g0, g1)
    isems = (si0, si1)
    gsems = (sg0, sg1)
    c = lax.axis_index("c")
    s = lax.axis_index("s")
    wid = s * NC + c
    chrow0 = wid * CPW

    def idx_copy(st, p):
        pltpu.async_copy(idx_hbm.at[pl.ds(chrow0 + st * NCHS, NCHS), :],
                         idxs[p], isems[p])

    def idx_wait(p):
        pltpu.make_async_copy(idx_hbm.at[pl.ds(0, NCHS), :], idxs[p],
                              isems[p]).wait()

    def gathers(p):
        for ch in range(NCHS):
            pltpu.async_copy(table_hbm.at[idxs[p].at[ch]],
                             gbufs[p].at[pl.ds(ch * GCH, GCH)], gsems[p])

    def gathers_wait(p):
        pltpu.make_async_copy(out_hbm.at[pl.ds(0, IPS)], gbufs[p],
                              gsems[p]).wait()

    def accum(p, st):
        buf = gbufs[p]

        def row_body(r, carry):
            base = r * SEQLEN
            parts = [buf[base + j, :] for j in range(NACC)]
            for j in range(NACC, SEQLEN):
                parts[j % NACC] = parts[j % NACC] + buf[base + j, :]
            while len(parts) > 1:
                parts = [parts[k] + parts[k + 1]
                         for k in range(0, len(parts), 2)]
            out_v[st * RPS + r, :] = parts[0]
            return carry

        lax.fori_loop(0, RPS, row_body, 0, unroll=False)

    idx_copy(0, 0)
    idx_wait(0)
    gathers(0)
    idx_copy(1, 1)

    def outer(s2, carry):
        for p in range(2):
            st = 2 * s2 + p
            gathers_wait(p)

            @pl.when(st + 2 < NSTEP)
            def _():
                idx_copy(st + 2, p)

            @pl.when(st + 1 < NSTEP)
            def _():
                idx_wait(1 - p)
                gathers(1 - p)

            accum(p, st)
        return carry

    lax.fori_loop(0, NSTEP // 2, outer, 0, unroll=False)

    pltpu.sync_copy(out_v, out_hbm.at[pl.ds(wid * BPW, BPW)])


@jax.jit
def _pool(idx, table):
    mesh = plsc.VectorSubcoreMesh(
        core_axis_name="c", subcore_axis_name="s",
        num_cores=NC, num_subcores=NS)
    f = pl.kernel(
        _pool_body,
        out_type=jax.ShapeDtypeStruct((BATCH, EMBED), jnp.float32),
        mesh=mesh,
        compiler_params=pltpu.CompilerParams(use_tc_tiling_on_sc=False),
        scratch_types=[
            pltpu.VMEM((NCHS, GCH), jnp.int32),
            pltpu.VMEM((NCHS, GCH), jnp.int32),
            pltpu.VMEM((IPS, EMBED), jnp.float32),
            pltpu.VMEM((IPS, EMBED), jnp.float32),
            pltpu.VMEM((BPW, EMBED), jnp.float32),
            pltpu.SemaphoreType.DMA,
            pltpu.SemaphoreType.DMA,
            pltpu.SemaphoreType.DMA,
            pltpu.SemaphoreType.DMA,
        ],
    )
    return f(idx, table)


def _mlp_body(x_ref, w1_ref, b1_ref, w2_ref, b2_ref, o_ref):
    x = x_ref[...]
    h = jnp.dot(x, w1_ref[...], preferred_element_type=jnp.float32) + b1_ref[...]
    h = jnp.maximum(h, 0.0)
    z = jnp.dot(h, w2_ref[...], preferred_element_type=jnp.float32) + b2_ref[...]
    o_ref[...] = 1.0 / (1.0 + jnp.exp(-z))


@jax.jit
def _mlp(pooled, w1s, b1, w2, b2):
    return pl.pallas_call(
        _mlp_body,
        out_shape=jax.ShapeDtypeStruct((BATCH, 1), jnp.float32),
    )(pooled, w1s, b1, w2, b2)


def kernel(indices, table, W1, b1, W2, b2):
    idx = indices.astype(jnp.int32).reshape(NW * CPW, GCH)
    pooled = _pool(idx, table)
    return _mlp(pooled, W1 * (1.0 / SEQLEN), b1.reshape(1, HIDDEN),
                W2, b2.reshape(1, 1))
